# baseline (speedup 1.0000x reference)
import functools

import jax
import jax.numpy as jnp
from jax import lax
from jax.experimental import pallas as pl
from jax.experimental.pallas import tpu as pltpu
from jax.experimental.pallas import tpu_sc as plsc

B = 16384
HIST = 50
D = 32
GROUP = 128 // D
NG = 1000000 // GROUP
NC = 2
NS = 16
NW = NC * NS
PB = B // NW
C = 8
NCH = PB // C
RPC = C * HIST
GSUB = 80
NSUB = RPC // GSUB
AVG_RATING = 3.0


_GDN = lax.GatherDimensionNumbers(
    offset_dims=(), collapsed_slice_dims=(0,), start_index_map=(0,))


def _permute(x, idx):
    return lax.gather(x, idx[:, None], _GDN, (1,),
                      mode=lax.GatherScatterMode.PROMISE_IN_BOUNDS)


def _hsum(x, iota):
    for sh in (1, 2, 4, 8):
        x = x + _permute(x, iota ^ sh)
    return x


def _rsqrt(x):
    y = (0.5 * jnp.where(x >= 4.0, 0.5, 1.0)
         * jnp.where(x >= 16.0, 0.5, 1.0))
    for _ in range(6):
        y = y * (1.5 - 0.5 * x * y * y)
    return y


@functools.partial(
    pl.kernel,
    out_type=(
        jax.ShapeDtypeStruct((B,), jnp.float32),
        jax.ShapeDtypeStruct((B,), jnp.float32),
        jax.ShapeDtypeStruct((B,), jnp.float32),
    ),
    mesh=plsc.VectorSubcoreMesh(core_axis_name="c", subcore_axis_name="s"),
    scratch_types=[
        pltpu.VMEM((RPC,), jnp.int32),
        pltpu.VMEM((NSUB, GSUB), jnp.int32),
        pltpu.VMEM((RPC,), jnp.int32),
        pltpu.VMEM((RPC, 128), jnp.float32),
        pltpu.VMEM((16,), jnp.int32),
        pltpu.VMEM((16,), jnp.int32),
        pltpu.VMEM((16,), jnp.int32),
        pltpu.VMEM((16,), jnp.int32),
        pltpu.VMEM((16,), jnp.int32),
        pltpu.VMEM((16,), jnp.int32),
        pltpu.VMEM((16, 128), jnp.float32),
        pltpu.VMEM((16, 128), jnp.float32),
        pltpu.VMEM((16,), jnp.float32),
        pltpu.VMEM((16,), jnp.float32),
        pltpu.VMEM((1, 128), jnp.float32),
        pltpu.VMEM((PB + 16 - C,), jnp.float32),
        pltpu.VMEM((PB + 16 - C,), jnp.float32),
        pltpu.VMEM((PB + 16 - C,), jnp.float32),
        pltpu.SemaphoreType.DMA,
        pltpu.SemaphoreType.DMA,
    ],
)
def _svdpp(user_h, item_h, simf_h, ub_h, ib_h, iq_h, up_h, iy_h,
           out_h, ubo_h, ibo_h,
           sflat, g2, qoff, rows, uidx, iidx, ugi, igi, uqo, iqo,
           upc, iqc, ubc, ibc, y0, outv, ubov, ibov, sem_r, sem_s):
    wid = lax.axis_index("s") * NC + lax.axis_index("c")
    base = wid * PB
    iota = lax.iota(jnp.int32, 16)
    mtail = iota >= 14

    pltpu.sync_copy(iy_h.at[pl.ds(0, 1)], y0)
    y00 = y0[0, pl.ds(0, 16)]
    y01 = y0[0, pl.ds(16, 16)]

    def chunk(g, carry):
        cb = pl.multiple_of(base + g * C, C)
        pltpu.sync_copy(simf_h.at[pl.ds(pl.multiple_of(cb * HIST, RPC), RPC)],
                        sflat)
        pltpu.sync_copy(user_h.at[pl.ds(cb, C)], uidx.at[pl.ds(0, C)])
        pltpu.sync_copy(item_h.at[pl.ds(cb, C)], iidx.at[pl.ds(0, C)])
        for j in range(RPC // 16):
            p = j * 16
            v = sflat[pl.ds(p, 16)]
            g2[p // GSUB, pl.ds(p % GSUB, 16)] = v >> 2
            qoff[pl.ds(p, 16)] = (v & 3) * D
        uv = jnp.where(iota < C, uidx[...], 0)
        iv = jnp.where(iota < C, iidx[...], 0)
        uidx[...] = uv
        iidx[...] = iv
        ugi[...] = uv >> 2
        igi[...] = iv >> 2
        uqo[...] = (uv & 3) * D
        iqo[...] = (iv & 3) * D
        cps = []
        for j in range(NSUB):
            cps.append(pltpu.async_copy(
                iy_h.at[g2.at[j]], rows.at[pl.ds(j * GSUB, GSUB), :], sem_r))
        cps.append(pltpu.async_copy(up_h.at[ugi], upc, sem_s))
        cps.append(pltpu.async_copy(iq_h.at[igi], iqc, sem_s))
        cps.append(pltpu.async_copy(ub_h.at[uidx], ubc, sem_s))
        cps.append(pltpu.async_copy(ib_h.at[iidx], ibc, sem_s))

        cnt = jnp.zeros((16,), jnp.float32)
        for b in range(C):
            p = b * HIST
            v0 = sflat[pl.ds(p, 16)]
            v1 = sflat[pl.ds(p + 16, 16)]
            v2 = sflat[pl.ds(p + 32, 16)]
            v3 = sflat[pl.ds(p + 34, 16)]
            z = (jnp.where(v0 == 0, 1.0, 0.0)
                 + jnp.where(v1 == 0, 1.0, 0.0)
                 + jnp.where(v2 == 0, 1.0, 0.0)
                 + jnp.where((v3 == 0) & mtail, 1.0, 0.0))
            cnt = jnp.where(iota == b, _hsum(z, iota), cnt)
        neff = 50.0 - cnt
        inv = 1.0 / (neff * _rsqrt(neff) + 1e-13)
        inv = jnp.where(neff == 0.0, 0.0, inv)

        for cp in cps:
            cp.wait()

        uqv = uqo[...]
        iqv = iqo[...]
        tot = jnp.zeros((16,), jnp.float32)
        for b in range(C):
            fb = jnp.full((16,), b, jnp.int32)
            p = b * HIST
            qv = (qoff[pl.ds(p, 16)], qoff[pl.ds(p + 16, 16)],
                  qoff[pl.ds(p + 32, 16)], qoff[pl.ds(p + 34, 16)])
            a0 = jnp.zeros((16,), jnp.float32)
            a1 = jnp.zeros((16,), jnp.float32)
            for n in range(HIST):
                r = p + n
                q = qv[3][n - 34] if n >= 48 else qv[n // 16][n % 16]
                a0 = a0 + rows[r, pl.ds(q, 16)]
                a1 = a1 + rows[r, pl.ds(q + 16, 16)]
            c0 = _permute(cnt, fb)
            ivn = _permute(inv, fb)
            s0 = (a0 - c0 * y00) * ivn
            s1 = (a1 - c0 * y01) * ivn
            uo = uqv[b]
            io = iqv[b]
            u0 = upc[b, pl.ds(uo, 16)]
            u1 = upc[b, pl.ds(uo + 16, 16)]
            q0 = iqc[b, pl.ds(io, 16)]
            q1 = iqc[b, pl.ds(io + 16, 16)]
            prod = (u0 + s0) * q0 + (u1 + s1) * q1
            tot = jnp.where(iota == b, _hsum(prod, iota), tot)

        ubv = ubc[...]
        ibv = ibc[...]
        off = g * C
        ubov[pl.ds(off, 16)] = ubv
        ibov[pl.ds(off, 16)] = ibv
        outv[pl.ds(off, 16)] = AVG_RATING + ubv + ibv + tot
        return carry

    lax.fori_loop(0, NCH, chunk, 0)
    pltpu.sync_copy(outv.at[pl.ds(0, PB)], out_h.at[pl.ds(base, PB)])
    pltpu.sync_copy(ubov.at[pl.ds(0, PB)], ubo_h.at[pl.ds(base, PB)])
    pltpu.sync_copy(ibov.at[pl.ds(0, PB)], ibo_h.at[pl.ds(base, PB)])


def kernel(user, item, similar_implicit, user_bias, item_bias, item_q,
           user_p, item_y):
    simf = similar_implicit.reshape(B * HIST)
    out, ub, ib = _svdpp(user, item, simf, user_bias, item_bias,
                         item_q.reshape(NG, 128), user_p.reshape(NG, 128),
                         item_y.reshape(NG, 128))
    return (out, ub, ib)


# speedup vs baseline: 1.7075x; 1.7075x over previous
import functools

import jax
import jax.numpy as jnp
from jax import lax
from jax.experimental import pallas as pl
from jax.experimental import layout as jex_layout
from jax.experimental.pallas import tpu as pltpu
from jax.experimental.pallas import tpu_sc as plsc

B = 16384
HIST = 50
D = 32
NC = 2
NS = 16
NW = NC * NS
PB = B // NW
C = 16
NCH = PB // C
RPC = C * HIST
GSUB = 80
NSUB = RPC // GSUB
AVG_RATING = 3.0


_GDN = lax.GatherDimensionNumbers(
    offset_dims=(), collapsed_slice_dims=(0,), start_index_map=(0,))


def _permute(x, idx):
    return lax.gather(x, idx[:, None], _GDN, (1,),
                      mode=lax.GatherScatterMode.PROMISE_IN_BOUNDS)


def _hsum(x, iota):
    for sh in (1, 2, 4, 8):
        x = x + _permute(x, iota ^ sh)
    return x


def _rsqrt(x):
    y = (0.5 * jnp.where(x >= 4.0, 0.5, 1.0)
         * jnp.where(x >= 16.0, 0.5, 1.0))
    for _ in range(6):
        y = y * (1.5 - 0.5 * x * y * y)
    return y


@functools.partial(
    pl.kernel,
    out_type=(
        jax.ShapeDtypeStruct((B,), jnp.float32),
        jax.ShapeDtypeStruct((B,), jnp.float32),
        jax.ShapeDtypeStruct((B,), jnp.float32),
    ),
    mesh=plsc.VectorSubcoreMesh(core_axis_name="c", subcore_axis_name="s"),
    compiler_params=pltpu.CompilerParams(use_tc_tiling_on_sc=False),
    scratch_types=[
        pltpu.VMEM((NSUB, GSUB), jnp.int32),
        pltpu.VMEM((RPC,), jnp.int32),
        pltpu.VMEM((RPC, D), jnp.float32),
        pltpu.VMEM((C,), jnp.int32),
        pltpu.VMEM((C,), jnp.int32),
        pltpu.VMEM((C, D), jnp.float32),
        pltpu.VMEM((C, D), jnp.float32),
        pltpu.VMEM((C,), jnp.float32),
        pltpu.VMEM((C,), jnp.float32),
        pltpu.VMEM((1, D), jnp.float32),
        pltpu.VMEM((PB,), jnp.float32),
        pltpu.VMEM((PB,), jnp.float32),
        pltpu.VMEM((PB,), jnp.float32),
        pltpu.SemaphoreType.DMA,
        pltpu.SemaphoreType.DMA,
    ],
)
def _svdpp(user_h, item_h, simf_h, ub_h, ib_h, iq_h, up_h, iy_h,
           out_h, ubo_h, ibo_h,
           g2, sflat, rows, uidx, iidx, upc, iqc, ubc, ibc,
           y0, outv, ubov, ibov, sem_r, sem_s):
    wid = lax.axis_index("s") * NC + lax.axis_index("c")
    base = wid * PB
    iota = lax.iota(jnp.int32, 16)
    mtail = iota >= 14

    pltpu.sync_copy(iy_h.at[pl.ds(0, 1)], y0)
    y00 = y0[0, pl.ds(0, 16)]
    y01 = y0[0, pl.ds(16, 16)]

    def chunk(g, carry):
        cb = pl.multiple_of(base + g * C, C)
        pltpu.sync_copy(simf_h.at[pl.ds(pl.multiple_of(cb * HIST, RPC), RPC)],
                        sflat)
        pltpu.sync_copy(user_h.at[pl.ds(cb, C)], uidx)
        pltpu.sync_copy(item_h.at[pl.ds(cb, C)], iidx)
        for j in range(RPC // 16):
            p = j * 16
            g2[p // GSUB, pl.ds(p % GSUB, 16)] = sflat[pl.ds(p, 16)]
        cps = []
        for j in range(NSUB):
            cps.append(pltpu.async_copy(
                iy_h.at[g2.at[j]], rows.at[pl.ds(j * GSUB, GSUB), :], sem_r))
        cps.append(pltpu.async_copy(up_h.at[uidx], upc, sem_s))
        cps.append(pltpu.async_copy(iq_h.at[iidx], iqc, sem_s))
        cps.append(pltpu.async_copy(ub_h.at[uidx], ubc, sem_s))
        cps.append(pltpu.async_copy(ib_h.at[iidx], ibc, sem_s))

        cnt = jnp.zeros((16,), jnp.float32)
        for b in range(C):
            p = b * HIST
            v0 = sflat[pl.ds(p, 16)]
            v1 = sflat[pl.ds(p + 16, 16)]
            v2 = sflat[pl.ds(p + 32, 16)]
            v3 = sflat[pl.ds(p + 34, 16)]
            z = (jnp.where(v0 == 0, 1.0, 0.0)
                 + jnp.where(v1 == 0, 1.0, 0.0)
                 + jnp.where(v2 == 0, 1.0, 0.0)
                 + jnp.where((v3 == 0) & mtail, 1.0, 0.0))
            cnt = jnp.where(iota == b, _hsum(z, iota), cnt)
        neff = 50.0 - cnt
        inv = 1.0 / (neff * _rsqrt(neff) + 1e-13)
        inv = jnp.where(neff == 0.0, 0.0, inv)

        for cp in cps:
            cp.wait()

        tot = jnp.zeros((16,), jnp.float32)
        for b in range(C):
            fb = jnp.full((16,), b, jnp.int32)
            a0 = jnp.zeros((16,), jnp.float32)
            a1 = jnp.zeros((16,), jnp.float32)
            for n in range(HIST):
                r = b * HIST + n
                a0 = a0 + rows[r, pl.ds(0, 16)]
                a1 = a1 + rows[r, pl.ds(16, 16)]
            c0 = _permute(cnt, fb)
            ivn = _permute(inv, fb)
            s0 = (a0 - c0 * y00) * ivn
            s1 = (a1 - c0 * y01) * ivn
            u0 = upc[b, pl.ds(0, 16)]
            u1 = upc[b, pl.ds(16, 16)]
            q0 = iqc[b, pl.ds(0, 16)]
            q1 = iqc[b, pl.ds(16, 16)]
            prod = (u0 + s0) * q0 + (u1 + s1) * q1
            tot = jnp.where(iota == b, _hsum(prod, iota), tot)

        ubv = ubc[...]
        ibv = ibc[...]
        off = g * C
        ubov[pl.ds(off, C)] = ubv
        ibov[pl.ds(off, C)] = ibv
        outv[pl.ds(off, C)] = AVG_RATING + ubv + ibv + tot
        return carry

    lax.fori_loop(0, NCH, chunk, 0)
    pltpu.sync_copy(outv, out_h.at[pl.ds(base, PB)])
    pltpu.sync_copy(ubov, ubo_h.at[pl.ds(base, PB)])
    pltpu.sync_copy(ibov, ibo_h.at[pl.ds(base, PB)])


def _linear(t):
    return jex_layout.with_layout_constraint(
        t, jex_layout.Layout(tuple(range(t.ndim)), tiling=()))


def kernel(user, item, similar_implicit, user_bias, item_bias, item_q,
           user_p, item_y):
    simf = similar_implicit.reshape(B * HIST)
    out, ub, ib = _svdpp(user, item, simf, user_bias, item_bias,
                         _linear(item_q), _linear(user_p), _linear(item_y))
    return (out, ub, ib)
